# Initial kernel scaffold; baseline (speedup 1.0000x reference)
import jax
import jax.numpy as jnp
from jax.experimental import pallas as pl


def kernel(x, edge_index, batch, W1, b1, W2, b2, fcW, fcb):
    raise NotImplementedError("write your pallas kernel here")


# speedup vs baseline: 13.3050x; 13.3050x over previous
import functools
import jax
import jax.numpy as jnp
from jax import lax
from jax.experimental import pallas as pl
from jax.experimental.pallas import tpu as pltpu, tpu_sc as plsc

N_NODES = 50000
NUM_GRAPHS = 128
NPAD = 50176
DUMMY = NPAD - 1
EPAD = 819200
B = 128
NC, NS = 2, 16
ROWS_PER_SUB = NPAD // NS

_mesh = lambda: plsc.VectorSubcoreMesh(core_axis_name="c", subcore_axis_name="s")
_SC_PARAMS = pltpu.CompilerParams(use_tc_tiling_on_sc=False)


def _deg_body(dst_h, ones_h, zeros_h, out_h, ones_v, didx_v, acc, sem):
    c = lax.axis_index("c")
    s = lax.axis_index("s")
    w = c * NS + s
    pltpu.sync_copy(zeros_h.at[pl.ds(s * ROWS_PER_SUB, ROWS_PER_SUB)],
                    acc.at[pl.ds(s * ROWS_PER_SUB, ROWS_PER_SUB)])
    pltpu.sync_copy(ones_h, ones_v)
    plsc.subcore_barrier()
    epw = EPAD // (NC * NS)

    def body(i, carry):
        base = w * epw + i * B
        pltpu.sync_copy(dst_h.at[pl.ds(base, B)], didx_v)
        pltpu.sync_copy(ones_v, acc.at[didx_v], add=True)
        return carry

    lax.fori_loop(0, epw // B, body, 0)
    plsc.subcore_barrier()
    pltpu.sync_copy(acc.at[pl.ds(s * ROWS_PER_SUB, ROWS_PER_SUB)],
                    out_h.at[c, pl.ds(s * ROWS_PER_SUB, ROWS_PER_SUB)])


def _deg_call(dstp, ones16, zeros16):
    return pl.kernel(
        _deg_body,
        out_type=jax.ShapeDtypeStruct((NC, NPAD, 16), jnp.float32),
        mesh=_mesh(),
        compiler_params=_SC_PARAMS,
        scratch_types=[
            pltpu.VMEM((B, 16), jnp.float32),
            pltpu.VMEM((B,), jnp.int32),
            pltpu.VMEM_SHARED((NPAD, 16), jnp.float32),
            pltpu.SemaphoreType.DMA,
        ],
    )(dstp, ones16, zeros16)


def _agg1_body(src_h, dst_h, tbl_h, zeros_h, out_h, sidx_v, didx_v, rows_v,
               acc, sem):
    c = lax.axis_index("c")
    s = lax.axis_index("s")
    w = c * NS + s
    pltpu.sync_copy(zeros_h.at[pl.ds(s * ROWS_PER_SUB, ROWS_PER_SUB)],
                    acc.at[pl.ds(s * ROWS_PER_SUB, ROWS_PER_SUB)])
    plsc.subcore_barrier()
    epw = EPAD // (NC * NS)

    def body(i, carry):
        base = w * epw + i * B
        pltpu.sync_copy(src_h.at[pl.ds(base, B)], sidx_v)
        pltpu.sync_copy(dst_h.at[pl.ds(base, B)], didx_v)
        pltpu.async_copy(tbl_h.at[sidx_v], rows_v, sem).wait()
        pltpu.sync_copy(rows_v, acc.at[didx_v], add=True)
        return carry

    lax.fori_loop(0, epw // B, body, 0)
    plsc.subcore_barrier()
    pltpu.sync_copy(acc.at[pl.ds(s * ROWS_PER_SUB, ROWS_PER_SUB)],
                    out_h.at[c, pl.ds(s * ROWS_PER_SUB, ROWS_PER_SUB)])


def _agg1_call(srcp, dstp, xs16, zeros16):
    return pl.kernel(
        _agg1_body,
        out_type=jax.ShapeDtypeStruct((NC, NPAD, 16), jnp.float32),
        mesh=_mesh(),
        compiler_params=_SC_PARAMS,
        scratch_types=[
            pltpu.VMEM((B,), jnp.int32),
            pltpu.VMEM((B,), jnp.int32),
            pltpu.VMEM((B, 16), jnp.float32),
            pltpu.VMEM_SHARED((NPAD, 16), jnp.float32),
            pltpu.SemaphoreType.DMA,
        ],
    )(srcp, dstp, xs16, zeros16)


def _agg2_body(srcoff_h, dst_h, tbl_h, zeros_h, out_h, sidx_v, didx_v, rows_v,
               acc, sem):
    c = lax.axis_index("c")
    s = lax.axis_index("s")
    pltpu.sync_copy(zeros_h.at[pl.ds(s * ROWS_PER_SUB, ROWS_PER_SUB)],
                    acc.at[pl.ds(s * ROWS_PER_SUB, ROWS_PER_SUB)])
    plsc.subcore_barrier()
    epw = EPAD // NS

    def body(i, carry):
        base = s * epw + i * B
        pltpu.sync_copy(srcoff_h.at[c, pl.ds(base, B)], sidx_v)
        pltpu.sync_copy(dst_h.at[pl.ds(base, B)], didx_v)
        pltpu.async_copy(tbl_h.at[sidx_v], rows_v, sem).wait()
        pltpu.sync_copy(rows_v, acc.at[didx_v], add=True)
        return carry

    lax.fori_loop(0, epw // B, body, 0)
    plsc.subcore_barrier()
    pltpu.sync_copy(acc.at[pl.ds(s * ROWS_PER_SUB, ROWS_PER_SUB)],
                    out_h.at[c, pl.ds(s * ROWS_PER_SUB, ROWS_PER_SUB)])


def _agg2_call(srcoff, dstp, hsflat, zeros32):
    return pl.kernel(
        _agg2_body,
        out_type=jax.ShapeDtypeStruct((NC, NPAD, 32), jnp.float32),
        mesh=_mesh(),
        compiler_params=_SC_PARAMS,
        scratch_types=[
            pltpu.VMEM((B,), jnp.int32),
            pltpu.VMEM((B,), jnp.int32),
            pltpu.VMEM((B, 32), jnp.float32),
            pltpu.VMEM_SHARED((NPAD, 32), jnp.float32),
            pltpu.SemaphoreType.DMA,
        ],
    )(srcoff, dstp, hsflat, zeros32)


RB = 6272
NBLK = NPAD // RB


def _prep_body(dacc, x16, dinv_o, xs16_o):
    deg = dacc[0, :, 0:1] + dacc[1, :, 0:1] + 1.0
    dinv = lax.rsqrt(deg)
    dinv_o[...] = dinv
    xs16_o[...] = x16[...] * dinv


def _prep_call(dacc, x16):
    return pl.pallas_call(
        _prep_body,
        grid=(NBLK,),
        in_specs=[
            pl.BlockSpec((NC, RB, 16), lambda i: (0, i, 0)),
            pl.BlockSpec((RB, 16), lambda i: (i, 0)),
        ],
        out_specs=[
            pl.BlockSpec((RB, 1), lambda i: (i, 0)),
            pl.BlockSpec((RB, 16), lambda i: (i, 0)),
        ],
        out_shape=[
            jax.ShapeDtypeStruct((NPAD, 1), jnp.float32),
            jax.ShapeDtypeStruct((NPAD, 16), jnp.float32),
        ],
    )(dacc, x16)


def _hidden_body(aacc, xs16, dinv, W1, b1, hs_o):
    s = (aacc[0] + aacc[1] + xs16[...]) * dinv[...]
    h = jnp.maximum(
        jax.lax.dot_general(s[:, 0:2], W1[...], (((1,), (0,)), ((), ())),
                            preferred_element_type=jnp.float32) + b1[...], 0.0)
    hs = h * dinv[...]
    hs_o[0] = hs[:, 0:32]
    hs_o[1] = hs[:, 32:64]


def _hidden_call(aacc, xs16, dinv, W1, b1):
    return pl.pallas_call(
        _hidden_body,
        grid=(NBLK,),
        in_specs=[
            pl.BlockSpec((NC, RB, 16), lambda i: (0, i, 0)),
            pl.BlockSpec((RB, 16), lambda i: (i, 0)),
            pl.BlockSpec((RB, 1), lambda i: (i, 0)),
            pl.BlockSpec((2, 64), lambda i: (0, 0)),
            pl.BlockSpec((1, 64), lambda i: (0, 0)),
        ],
        out_specs=pl.BlockSpec((NC, RB, 32), lambda i: (0, i, 0)),
        out_shape=jax.ShapeDtypeStruct((NC, NPAD, 32), jnp.float32),
    )(aacc, xs16, dinv, W1, b1)


def _head_body(eacc, hs2, dinv, batchp, W2, b2, fcW, fcb, out_o, pooled, cnt):
    i = pl.program_id(0)

    @pl.when(i == 0)
    def _init():
        pooled[...] = jnp.zeros_like(pooled)
        cnt[...] = jnp.zeros_like(cnt)

    a = jnp.concatenate([eacc[0] + hs2[0], eacc[1] + hs2[1]], axis=1)
    a = a * dinv[...]
    oh = (batchp[...] == lax.broadcasted_iota(jnp.int32, (1, NUM_GRAPHS), 1))
    oh = oh.astype(jnp.float32)
    pooled[...] += lax.dot_general(oh, a, (((0,), (0,)), ((), ())),
                                   preferred_element_type=jnp.float32)
    cnt[...] += lax.dot_general(oh, jnp.ones((RB, 1), jnp.float32),
                                (((0,), (0,)), ((), ())),
                                preferred_element_type=jnp.float32)

    @pl.when(i == NBLK - 1)
    def _fin():
        pm = pooled[...] / jnp.maximum(cnt[...], 1.0)
        z = lax.dot_general(pm, W2[...], (((1,), (0,)), ((), ())),
                            preferred_element_type=jnp.float32) + b2[...]
        z = lax.dot_general(z, fcW[...], (((1,), (0,)), ((), ())),
                            preferred_element_type=jnp.float32) + fcb[...]
        m = jnp.max(z, axis=1, keepdims=True)
        out_o[...] = z - m - jnp.log(jnp.sum(jnp.exp(z - m), axis=1,
                                             keepdims=True))


def _head_call(eacc, hs2, dinv, batchp, W2, b2, fcW, fcb):
    return pl.pallas_call(
        _head_body,
        grid=(NBLK,),
        in_specs=[
            pl.BlockSpec((NC, RB, 32), lambda i: (0, i, 0)),
            pl.BlockSpec((NC, RB, 32), lambda i: (0, i, 0)),
            pl.BlockSpec((RB, 1), lambda i: (i, 0)),
            pl.BlockSpec((RB, 1), lambda i: (i, 0)),
            pl.BlockSpec((64, 128), lambda i: (0, 0)),
            pl.BlockSpec((1, 128), lambda i: (0, 0)),
            pl.BlockSpec((128, 30), lambda i: (0, 0)),
            pl.BlockSpec((1, 30), lambda i: (0, 0)),
        ],
        out_specs=pl.BlockSpec((NUM_GRAPHS, 30), lambda i: (0, 0)),
        out_shape=jax.ShapeDtypeStruct((NUM_GRAPHS, 30), jnp.float32),
        scratch_shapes=[
            pltpu.VMEM((NUM_GRAPHS, 64), jnp.float32),
            pltpu.VMEM((NUM_GRAPHS, 1), jnp.float32),
        ],
    )(eacc, hs2, dinv, batchp, W2, b2, fcW, fcb)


@jax.jit
def kernel(x, edge_index, batch, W1, b1, W2, b2, fcW, fcb):
    src = edge_index[0].astype(jnp.int32)
    dst = edge_index[1].astype(jnp.int32)
    npad_extra = NPAD - N_NODES
    epad_extra = EPAD - src.shape[0]
    srcp = jnp.pad(src, (0, epad_extra))
    dstp = jnp.pad(dst, (0, epad_extra), constant_values=DUMMY)
    srcoff = jnp.stack([srcp, srcp + NPAD])
    x16 = jnp.pad(x, ((0, npad_extra), (0, 14)))
    batchp = jnp.pad(batch.astype(jnp.int32), (0, npad_extra),
                     constant_values=NUM_GRAPHS).reshape(NPAD, 1)
    ones16 = jnp.ones((B, 16), jnp.float32)
    zeros16 = jnp.zeros((NPAD, 16), jnp.float32)
    zeros32 = jnp.zeros((NPAD, 32), jnp.float32)
    b1r = b1.reshape(1, 64)
    b2r = b2.reshape(1, 128)
    fcbr = fcb.reshape(1, 30)

    dacc = _deg_call(dstp, ones16, zeros16)
    dinv, xs16 = _prep_call(dacc, x16)
    aacc = _agg1_call(srcp, dstp, xs16, zeros16)
    hs2 = _hidden_call(aacc, xs16, dinv, W1, b1r)
    hsflat = hs2.reshape(NC * NPAD, 32)
    eacc = _agg2_call(srcoff, dstp, hsflat, zeros32)
    return _head_call(eacc, hs2, dinv, batchp, W2, b2r, fcW, fcbr)


# speedup vs baseline: 25.8700x; 1.9444x over previous
import functools
import jax
import jax.numpy as jnp
from jax import lax
from jax.experimental import pallas as pl
from jax.experimental.pallas import tpu as pltpu, tpu_sc as plsc

N_NODES = 50000
NUM_GRAPHS = 128
NPAD = 50176
DUMMY = NPAD - 1
EPAD = 819200
B = 128
NBT = EPAD // B
NC, NS = 2, 16
ROWS_PER_SUB = NPAD // NS
NB1 = NBT // (NC * NS)
NB2 = NBT // NS

_mesh = lambda: plsc.VectorSubcoreMesh(core_axis_name="c", subcore_axis_name="s")
_SC_PARAMS = pltpu.CompilerParams(use_tc_tiling_on_sc=False)


def _zero_acc(zeros_h, acc, s):
    pltpu.sync_copy(zeros_h.at[pl.ds(s * ROWS_PER_SUB, ROWS_PER_SUB)],
                    acc.at[pl.ds(s * ROWS_PER_SUB, ROWS_PER_SUB)])


def _copy_out(acc, out_h, c, s):
    pltpu.sync_copy(acc.at[pl.ds(s * ROWS_PER_SUB, ROWS_PER_SUB)],
                    out_h.at[c, pl.ds(s * ROWS_PER_SUB, ROWS_PER_SUB)])


def _gather_scatter_loop(tbl_h, acc, sidx_all, didx_all, rows0, rows1,
                         sem0, sem1, nb):
    pltpu.async_copy(tbl_h.at[sidx_all.at[0]], rows0, sem0)
    pltpu.async_copy(tbl_h.at[sidx_all.at[1]], rows1, sem1)

    def body(t, carry):
        b = 2 * t
        pltpu.make_async_copy(tbl_h.at[sidx_all.at[b]], rows0, sem0).wait()
        pltpu.sync_copy(rows0, acc.at[didx_all.at[b]], add=True)
        pltpu.async_copy(tbl_h.at[sidx_all.at[b + 2]], rows0, sem0)
        pltpu.make_async_copy(tbl_h.at[sidx_all.at[b + 1]], rows1, sem1).wait()
        pltpu.sync_copy(rows1, acc.at[didx_all.at[b + 1]], add=True)
        pltpu.async_copy(tbl_h.at[sidx_all.at[b + 3]], rows1, sem1)
        return carry

    lax.fori_loop(0, nb // 2 - 1, body, 0)
    b = nb - 2
    pltpu.make_async_copy(tbl_h.at[sidx_all.at[b]], rows0, sem0).wait()
    pltpu.sync_copy(rows0, acc.at[didx_all.at[b]], add=True)
    pltpu.make_async_copy(tbl_h.at[sidx_all.at[b + 1]], rows1, sem1).wait()
    pltpu.sync_copy(rows1, acc.at[didx_all.at[b + 1]], add=True)


def _deg_body(dst_h, ones_h, zeros_h, out_h, ones_v, didx_all, acc, sem):
    c = lax.axis_index("c")
    s = lax.axis_index("s")
    w = c * NS + s
    _zero_acc(zeros_h, acc, s)
    pltpu.sync_copy(ones_h, ones_v)
    pltpu.sync_copy(dst_h.at[pl.ds(w * NB1, NB1)], didx_all)
    plsc.subcore_barrier()

    def body(i, carry):
        pltpu.sync_copy(ones_v, acc.at[didx_all.at[i]], add=True)
        return carry

    lax.fori_loop(0, NB1, body, 0)
    plsc.subcore_barrier()
    _copy_out(acc, out_h, c, s)


def _deg_call(dst2, ones16, zeros16):
    return pl.kernel(
        _deg_body,
        out_type=jax.ShapeDtypeStruct((NC, NPAD, 16), jnp.float32),
        mesh=_mesh(),
        compiler_params=_SC_PARAMS,
        scratch_types=[
            pltpu.VMEM((B, 16), jnp.float32),
            pltpu.VMEM((NB1, B), jnp.int32),
            pltpu.VMEM_SHARED((NPAD, 16), jnp.float32),
            pltpu.SemaphoreType.DMA,
        ],
    )(dst2, ones16, zeros16)


def _agg1_body(src_h, dst_h, tbl_h, zeros_h, out_h, sidx_all, didx_all,
               rows0, rows1, acc, sem0, sem1):
    c = lax.axis_index("c")
    s = lax.axis_index("s")
    w = c * NS + s
    _zero_acc(zeros_h, acc, s)
    pltpu.sync_copy(src_h.at[pl.ds(w * NB1, NB1)], sidx_all)
    pltpu.sync_copy(dst_h.at[pl.ds(w * NB1, NB1)], didx_all)
    plsc.subcore_barrier()
    _gather_scatter_loop(tbl_h, acc, sidx_all, didx_all, rows0, rows1,
                         sem0, sem1, NB1)
    plsc.subcore_barrier()
    _copy_out(acc, out_h, c, s)


def _agg1_call(src2, dst2, xs16, zeros16):
    return pl.kernel(
        _agg1_body,
        out_type=jax.ShapeDtypeStruct((NC, NPAD, 16), jnp.float32),
        mesh=_mesh(),
        compiler_params=_SC_PARAMS,
        scratch_types=[
            pltpu.VMEM((NB1, B), jnp.int32),
            pltpu.VMEM((NB1, B), jnp.int32),
            pltpu.VMEM((B, 16), jnp.float32),
            pltpu.VMEM((B, 16), jnp.float32),
            pltpu.VMEM_SHARED((NPAD, 16), jnp.float32),
            pltpu.SemaphoreType.DMA,
            pltpu.SemaphoreType.DMA,
        ],
    )(src2, dst2, xs16, zeros16)


CH2 = 50


def _agg2_body(srcoff_h, dst_h, tbl_h, zeros_h, out_h, sidx_ch, didx_ch,
               rows0, rows1, acc, sem0, sem1):
    c = lax.axis_index("c")
    s = lax.axis_index("s")
    _zero_acc(zeros_h, acc, s)
    plsc.subcore_barrier()

    def chunk(k, carry):
        base = s * NB2 + k * CH2
        pltpu.sync_copy(srcoff_h.at[c, pl.ds(base, CH2)], sidx_ch)
        pltpu.sync_copy(dst_h.at[pl.ds(base, CH2)], didx_ch)
        _gather_scatter_loop(tbl_h, acc, sidx_ch, didx_ch, rows0, rows1,
                             sem0, sem1, CH2)
        return carry

    lax.fori_loop(0, NB2 // CH2, chunk, 0)
    plsc.subcore_barrier()
    _copy_out(acc, out_h, c, s)


def _agg2_call(srcoff, dst2, hsflat, zeros32):
    return pl.kernel(
        _agg2_body,
        out_type=jax.ShapeDtypeStruct((NC, NPAD, 32), jnp.float32),
        mesh=_mesh(),
        compiler_params=_SC_PARAMS,
        scratch_types=[
            pltpu.VMEM((CH2, B), jnp.int32),
            pltpu.VMEM((CH2, B), jnp.int32),
            pltpu.VMEM((B, 32), jnp.float32),
            pltpu.VMEM((B, 32), jnp.float32),
            pltpu.VMEM_SHARED((NPAD, 32), jnp.float32),
            pltpu.SemaphoreType.DMA,
            pltpu.SemaphoreType.DMA,
        ],
    )(srcoff, dst2, hsflat, zeros32)


RB = 6272
NBLK = NPAD // RB


def _prep_body(dacc, x16, dinv_o, xs16_o):
    deg = dacc[0, :, 0:1] + dacc[1, :, 0:1] + 1.0
    dinv = lax.rsqrt(deg)
    dinv_o[...] = dinv
    xs16_o[...] = x16[...] * dinv


def _prep_call(dacc, x16):
    return pl.pallas_call(
        _prep_body,
        grid=(NBLK,),
        in_specs=[
            pl.BlockSpec((NC, RB, 16), lambda i: (0, i, 0)),
            pl.BlockSpec((RB, 16), lambda i: (i, 0)),
        ],
        out_specs=[
            pl.BlockSpec((RB, 1), lambda i: (i, 0)),
            pl.BlockSpec((RB, 16), lambda i: (i, 0)),
        ],
        out_shape=[
            jax.ShapeDtypeStruct((NPAD, 1), jnp.float32),
            jax.ShapeDtypeStruct((NPAD, 16), jnp.float32),
        ],
    )(dacc, x16)


def _hidden_body(aacc, xs16, dinv, W1, b1, hs_o):
    s = (aacc[0] + aacc[1] + xs16[...]) * dinv[...]
    h = jnp.maximum(
        jax.lax.dot_general(s[:, 0:2], W1[...], (((1,), (0,)), ((), ())),
                            preferred_element_type=jnp.float32) + b1[...], 0.0)
    hs = h * dinv[...]
    hs_o[0] = hs[:, 0:32]
    hs_o[1] = hs[:, 32:64]


def _hidden_call(aacc, xs16, dinv, W1, b1):
    return pl.pallas_call(
        _hidden_body,
        grid=(NBLK,),
        in_specs=[
            pl.BlockSpec((NC, RB, 16), lambda i: (0, i, 0)),
            pl.BlockSpec((RB, 16), lambda i: (i, 0)),
            pl.BlockSpec((RB, 1), lambda i: (i, 0)),
            pl.BlockSpec((2, 64), lambda i: (0, 0)),
            pl.BlockSpec((1, 64), lambda i: (0, 0)),
        ],
        out_specs=pl.BlockSpec((NC, RB, 32), lambda i: (0, i, 0)),
        out_shape=jax.ShapeDtypeStruct((NC, NPAD, 32), jnp.float32),
    )(aacc, xs16, dinv, W1, b1)


def _head_body(eacc, hs2, dinv, batchp, W2, b2, fcW, fcb, out_o, pooled, cnt):
    i = pl.program_id(0)

    @pl.when(i == 0)
    def _init():
        pooled[...] = jnp.zeros_like(pooled)
        cnt[...] = jnp.zeros_like(cnt)

    a = jnp.concatenate([eacc[0] + hs2[0], eacc[1] + hs2[1]], axis=1)
    a = a * dinv[...]
    oh = (batchp[...] == lax.broadcasted_iota(jnp.int32, (1, NUM_GRAPHS), 1))
    oh = oh.astype(jnp.float32)
    pooled[...] += lax.dot_general(oh, a, (((0,), (0,)), ((), ())),
                                   preferred_element_type=jnp.float32)
    cnt[...] += lax.dot_general(oh, jnp.ones((RB, 1), jnp.float32),
                                (((0,), (0,)), ((), ())),
                                preferred_element_type=jnp.float32)

    @pl.when(i == NBLK - 1)
    def _fin():
        pm = pooled[...] / jnp.maximum(cnt[...], 1.0)
        z = lax.dot_general(pm, W2[...], (((1,), (0,)), ((), ())),
                            preferred_element_type=jnp.float32) + b2[...]
        z = lax.dot_general(z, fcW[...], (((1,), (0,)), ((), ())),
                            preferred_element_type=jnp.float32) + fcb[...]
        m = jnp.max(z, axis=1, keepdims=True)
        out_o[...] = z - m - jnp.log(jnp.sum(jnp.exp(z - m), axis=1,
                                             keepdims=True))


def _head_call(eacc, hs2, dinv, batchp, W2, b2, fcW, fcb):
    return pl.pallas_call(
        _head_body,
        grid=(NBLK,),
        in_specs=[
            pl.BlockSpec((NC, RB, 32), lambda i: (0, i, 0)),
            pl.BlockSpec((NC, RB, 32), lambda i: (0, i, 0)),
            pl.BlockSpec((RB, 1), lambda i: (i, 0)),
            pl.BlockSpec((RB, 1), lambda i: (i, 0)),
            pl.BlockSpec((64, 128), lambda i: (0, 0)),
            pl.BlockSpec((1, 128), lambda i: (0, 0)),
            pl.BlockSpec((128, 30), lambda i: (0, 0)),
            pl.BlockSpec((1, 30), lambda i: (0, 0)),
        ],
        out_specs=pl.BlockSpec((NUM_GRAPHS, 30), lambda i: (0, 0)),
        out_shape=jax.ShapeDtypeStruct((NUM_GRAPHS, 30), jnp.float32),
        scratch_shapes=[
            pltpu.VMEM((NUM_GRAPHS, 64), jnp.float32),
            pltpu.VMEM((NUM_GRAPHS, 1), jnp.float32),
        ],
    )(eacc, hs2, dinv, batchp, W2, b2, fcW, fcb)


@jax.jit
def kernel(x, edge_index, batch, W1, b1, W2, b2, fcW, fcb):
    src = edge_index[0].astype(jnp.int32)
    dst = edge_index[1].astype(jnp.int32)
    npad_extra = NPAD - N_NODES
    epad_extra = EPAD - src.shape[0]
    srcp = jnp.pad(src, (0, epad_extra))
    dstp = jnp.pad(dst, (0, epad_extra), constant_values=DUMMY)
    src2 = srcp.reshape(NBT, B)
    dst2 = dstp.reshape(NBT, B)
    srcoff = jnp.stack([src2, src2 + NPAD])
    x16 = jnp.pad(x, ((0, npad_extra), (0, 14)))
    batchp = jnp.pad(batch.astype(jnp.int32), (0, npad_extra),
                     constant_values=NUM_GRAPHS).reshape(NPAD, 1)
    ones16 = jnp.ones((B, 16), jnp.float32)
    zeros16 = jnp.zeros((NPAD, 16), jnp.float32)
    zeros32 = jnp.zeros((NPAD, 32), jnp.float32)
    b1r = b1.reshape(1, 64)
    b2r = b2.reshape(1, 128)
    fcbr = fcb.reshape(1, 30)

    dacc = _deg_call(dst2, ones16, zeros16)
    dinv, xs16 = _prep_call(dacc, x16)
    aacc = _agg1_call(src2, dst2, xs16, zeros16)
    hs2 = _hidden_call(aacc, xs16, dinv, W1, b1r)
    hsflat = hs2.reshape(NC * NPAD, 32)
    eacc = _agg2_call(srcoff, dst2, hsflat, zeros32)
    return _head_call(eacc, hs2, dinv, batchp, W2, b2r, fcW, fcbr)


# speedup vs baseline: 26.0391x; 1.0065x over previous
import functools
import jax
import jax.numpy as jnp
from jax import lax
from jax.experimental import pallas as pl
from jax.experimental.pallas import tpu as pltpu, tpu_sc as plsc

N_NODES = 50000
NUM_GRAPHS = 128
NPAD = 50176
DUMMY = NPAD - 1
EPAD = 819200
B = 128
NBT = EPAD // B
NC, NS = 2, 16
ROWS_PER_SUB = NPAD // NS
NB1 = NBT // (NC * NS)
NB2 = NBT // NS

_mesh = lambda: plsc.VectorSubcoreMesh(core_axis_name="c", subcore_axis_name="s")
_SC_PARAMS = pltpu.CompilerParams(use_tc_tiling_on_sc=False)


def _zero_acc(zeros_h, acc, s):
    pltpu.sync_copy(zeros_h.at[pl.ds(s * ROWS_PER_SUB, ROWS_PER_SUB)],
                    acc.at[pl.ds(s * ROWS_PER_SUB, ROWS_PER_SUB)])


def _copy_out(acc, out_h, c, s):
    pltpu.sync_copy(acc.at[pl.ds(s * ROWS_PER_SUB, ROWS_PER_SUB)],
                    out_h.at[c, pl.ds(s * ROWS_PER_SUB, ROWS_PER_SUB)])


K = 8
K2 = 5


def _gs_pipe(tbl_h, acc, sidx, didx, slots, gsem, ssem, nb, k):
    for j in range(k):
        pltpu.async_copy(tbl_h.at[sidx.at[j]], slots.at[j], gsem.at[j])

    def rnd(r, carry):
        b0 = r * k
        for j in range(k):
            pltpu.make_async_copy(tbl_h.at[sidx.at[b0 + j]], slots.at[j],
                                  gsem.at[j]).wait()
            pltpu.async_copy(slots.at[j], acc.at[didx.at[b0 + j]], ssem.at[j],
                             add=True)
        for j in range(k):
            pltpu.make_async_copy(slots.at[j], acc.at[didx.at[b0 + j]],
                                  ssem.at[j]).wait()
            pltpu.async_copy(tbl_h.at[sidx.at[b0 + k + j]], slots.at[j],
                             gsem.at[j])
        return carry

    lax.fori_loop(0, nb // k - 1, rnd, 0)
    b0 = nb - k
    for j in range(k):
        pltpu.make_async_copy(tbl_h.at[sidx.at[b0 + j]], slots.at[j],
                              gsem.at[j]).wait()
        pltpu.async_copy(slots.at[j], acc.at[didx.at[b0 + j]], ssem.at[j],
                         add=True)
    for j in range(k):
        pltpu.make_async_copy(slots.at[j], acc.at[didx.at[b0 + j]],
                              ssem.at[j]).wait()


def _deg_body(dst_h, ones_h, zeros_h, out_h, ones_v, didx_all, acc,
              sem0, sem1):
    c = lax.axis_index("c")
    s = lax.axis_index("s")
    w = c * NS + s
    _zero_acc(zeros_h, acc, s)
    pltpu.sync_copy(ones_h, ones_v)
    pltpu.sync_copy(dst_h.at[pl.ds(w * NB1, NB1)], didx_all)
    plsc.subcore_barrier()

    for j in range(K):
        pltpu.async_copy(ones_v, acc.at[didx_all.at[j]], sem0, add=True)

    def body2(g, carry):
        b0 = 2 * g * K
        for j in range(K):
            pltpu.async_copy(ones_v, acc.at[didx_all.at[b0 + K + j]], sem1,
                             add=True)
        for j in range(K):
            pltpu.make_async_copy(ones_v, acc.at[didx_all.at[b0 + j]],
                                  sem0).wait()
        for j in range(K):
            pltpu.async_copy(ones_v, acc.at[didx_all.at[b0 + 2 * K + j]],
                             sem0, add=True)
        for j in range(K):
            pltpu.make_async_copy(ones_v, acc.at[didx_all.at[b0 + K + j]],
                                  sem1).wait()
        return carry

    ngroups = NB1 // K
    lax.fori_loop(0, (ngroups - 1) // 2, body2, 0)
    b0 = (ngroups - 1) * K
    for j in range(K):
        pltpu.make_async_copy(ones_v, acc.at[didx_all.at[b0 + j]],
                              sem0).wait()
    plsc.subcore_barrier()
    _copy_out(acc, out_h, c, s)


def _deg_call(dst2, ones16, zeros16):
    return pl.kernel(
        _deg_body,
        out_type=jax.ShapeDtypeStruct((NC, NPAD, 16), jnp.float32),
        mesh=_mesh(),
        compiler_params=_SC_PARAMS,
        scratch_types=[
            pltpu.VMEM((B, 16), jnp.float32),
            pltpu.VMEM((NB1, B), jnp.int32),
            pltpu.VMEM_SHARED((NPAD, 16), jnp.float32),
            pltpu.SemaphoreType.DMA,
            pltpu.SemaphoreType.DMA,
        ],
    )(dst2, ones16, zeros16)


def _agg1_body(src_h, dst_h, tbl_h, zeros_h, out_h, sidx_all, didx_all,
               slots, acc, gsem, ssem):
    c = lax.axis_index("c")
    s = lax.axis_index("s")
    w = c * NS + s
    _zero_acc(zeros_h, acc, s)
    pltpu.sync_copy(src_h.at[pl.ds(w * NB1, NB1)], sidx_all)
    pltpu.sync_copy(dst_h.at[pl.ds(w * NB1, NB1)], didx_all)
    plsc.subcore_barrier()
    _gs_pipe(tbl_h, acc, sidx_all, didx_all, slots, gsem, ssem, NB1, K)
    plsc.subcore_barrier()
    _copy_out(acc, out_h, c, s)


def _agg1_call(src2, dst2, xs16, zeros16):
    return pl.kernel(
        _agg1_body,
        out_type=jax.ShapeDtypeStruct((NC, NPAD, 16), jnp.float32),
        mesh=_mesh(),
        compiler_params=_SC_PARAMS,
        scratch_types=[
            pltpu.VMEM((NB1, B), jnp.int32),
            pltpu.VMEM((NB1, B), jnp.int32),
            pltpu.VMEM((K, B, 16), jnp.float32),
            pltpu.VMEM_SHARED((NPAD, 16), jnp.float32),
            pltpu.SemaphoreType.DMA((K,)),
            pltpu.SemaphoreType.DMA((K,)),
        ],
    )(src2, dst2, xs16, zeros16)


CH2 = 20


def _agg2_body(srcoff_h, dst_h, tbl_h, zeros_h, out_h, sidx_ch, didx_ch,
               slots, acc, gsem, ssem):
    c = lax.axis_index("c")
    s = lax.axis_index("s")
    _zero_acc(zeros_h, acc, s)
    plsc.subcore_barrier()

    def chunk(k, carry):
        base = s * NB2 + k * CH2
        pltpu.sync_copy(srcoff_h.at[c, pl.ds(base, CH2)], sidx_ch)
        pltpu.sync_copy(dst_h.at[pl.ds(base, CH2)], didx_ch)
        _gs_pipe(tbl_h, acc, sidx_ch, didx_ch, slots, gsem, ssem, CH2, K2)
        return carry

    lax.fori_loop(0, NB2 // CH2, chunk, 0)
    plsc.subcore_barrier()
    _copy_out(acc, out_h, c, s)


def _agg2_call(srcoff, dst2, hsflat, zeros32):
    return pl.kernel(
        _agg2_body,
        out_type=jax.ShapeDtypeStruct((NC, NPAD, 32), jnp.float32),
        mesh=_mesh(),
        compiler_params=_SC_PARAMS,
        scratch_types=[
            pltpu.VMEM((CH2, B), jnp.int32),
            pltpu.VMEM((CH2, B), jnp.int32),
            pltpu.VMEM((K2, B, 32), jnp.float32),
            pltpu.VMEM_SHARED((NPAD, 32), jnp.float32),
            pltpu.SemaphoreType.DMA((K2,)),
            pltpu.SemaphoreType.DMA((K2,)),
        ],
    )(srcoff, dst2, hsflat, zeros32)


RB = 6272
NBLK = NPAD // RB


def _prep_body(dacc, x16, dinv_o, xs16_o):
    deg = dacc[0, :, 0:1] + dacc[1, :, 0:1] + 1.0
    dinv = lax.rsqrt(deg)
    dinv_o[...] = dinv
    xs16_o[...] = x16[...] * dinv


def _prep_call(dacc, x16):
    return pl.pallas_call(
        _prep_body,
        grid=(NBLK,),
        in_specs=[
            pl.BlockSpec((NC, RB, 16), lambda i: (0, i, 0)),
            pl.BlockSpec((RB, 16), lambda i: (i, 0)),
        ],
        out_specs=[
            pl.BlockSpec((RB, 1), lambda i: (i, 0)),
            pl.BlockSpec((RB, 16), lambda i: (i, 0)),
        ],
        out_shape=[
            jax.ShapeDtypeStruct((NPAD, 1), jnp.float32),
            jax.ShapeDtypeStruct((NPAD, 16), jnp.float32),
        ],
    )(dacc, x16)


def _hidden_body(aacc, xs16, dinv, W1, b1, hs_o):
    s = (aacc[0] + aacc[1] + xs16[...]) * dinv[...]
    h = jnp.maximum(
        jax.lax.dot_general(s[:, 0:2], W1[...], (((1,), (0,)), ((), ())),
                            preferred_element_type=jnp.float32) + b1[...], 0.0)
    hs = h * dinv[...]
    hs_o[0] = hs[:, 0:32]
    hs_o[1] = hs[:, 32:64]


def _hidden_call(aacc, xs16, dinv, W1, b1):
    return pl.pallas_call(
        _hidden_body,
        grid=(NBLK,),
        in_specs=[
            pl.BlockSpec((NC, RB, 16), lambda i: (0, i, 0)),
            pl.BlockSpec((RB, 16), lambda i: (i, 0)),
            pl.BlockSpec((RB, 1), lambda i: (i, 0)),
            pl.BlockSpec((2, 64), lambda i: (0, 0)),
            pl.BlockSpec((1, 64), lambda i: (0, 0)),
        ],
        out_specs=pl.BlockSpec((NC, RB, 32), lambda i: (0, i, 0)),
        out_shape=jax.ShapeDtypeStruct((NC, NPAD, 32), jnp.float32),
    )(aacc, xs16, dinv, W1, b1)


def _head_body(eacc, hs2, dinv, batchp, W2, b2, fcW, fcb, out_o, pooled, cnt):
    i = pl.program_id(0)

    @pl.when(i == 0)
    def _init():
        pooled[...] = jnp.zeros_like(pooled)
        cnt[...] = jnp.zeros_like(cnt)

    a = jnp.concatenate([eacc[0] + hs2[0], eacc[1] + hs2[1]], axis=1)
    a = a * dinv[...]
    oh = (batchp[...] == lax.broadcasted_iota(jnp.int32, (1, NUM_GRAPHS), 1))
    oh = oh.astype(jnp.float32)
    pooled[...] += lax.dot_general(oh, a, (((0,), (0,)), ((), ())),
                                   preferred_element_type=jnp.float32)
    cnt[...] += lax.dot_general(oh, jnp.ones((RB, 1), jnp.float32),
                                (((0,), (0,)), ((), ())),
                                preferred_element_type=jnp.float32)

    @pl.when(i == NBLK - 1)
    def _fin():
        pm = pooled[...] / jnp.maximum(cnt[...], 1.0)
        z = lax.dot_general(pm, W2[...], (((1,), (0,)), ((), ())),
                            preferred_element_type=jnp.float32) + b2[...]
        z = lax.dot_general(z, fcW[...], (((1,), (0,)), ((), ())),
                            preferred_element_type=jnp.float32) + fcb[...]
        m = jnp.max(z, axis=1, keepdims=True)
        out_o[...] = z - m - jnp.log(jnp.sum(jnp.exp(z - m), axis=1,
                                             keepdims=True))


def _head_call(eacc, hs2, dinv, batchp, W2, b2, fcW, fcb):
    return pl.pallas_call(
        _head_body,
        grid=(NBLK,),
        in_specs=[
            pl.BlockSpec((NC, RB, 32), lambda i: (0, i, 0)),
            pl.BlockSpec((NC, RB, 32), lambda i: (0, i, 0)),
            pl.BlockSpec((RB, 1), lambda i: (i, 0)),
            pl.BlockSpec((RB, 1), lambda i: (i, 0)),
            pl.BlockSpec((64, 128), lambda i: (0, 0)),
            pl.BlockSpec((1, 128), lambda i: (0, 0)),
            pl.BlockSpec((128, 30), lambda i: (0, 0)),
            pl.BlockSpec((1, 30), lambda i: (0, 0)),
        ],
        out_specs=pl.BlockSpec((NUM_GRAPHS, 30), lambda i: (0, 0)),
        out_shape=jax.ShapeDtypeStruct((NUM_GRAPHS, 30), jnp.float32),
        scratch_shapes=[
            pltpu.VMEM((NUM_GRAPHS, 64), jnp.float32),
            pltpu.VMEM((NUM_GRAPHS, 1), jnp.float32),
        ],
    )(eacc, hs2, dinv, batchp, W2, b2, fcW, fcb)


@jax.jit
def kernel(x, edge_index, batch, W1, b1, W2, b2, fcW, fcb):
    src = edge_index[0].astype(jnp.int32)
    dst = edge_index[1].astype(jnp.int32)
    npad_extra = NPAD - N_NODES
    epad_extra = EPAD - src.shape[0]
    srcp = jnp.pad(src, (0, epad_extra))
    dstp = jnp.pad(dst, (0, epad_extra), constant_values=DUMMY)
    src2 = srcp.reshape(NBT, B)
    dst2 = dstp.reshape(NBT, B)
    srcoff = jnp.stack([src2, src2 + NPAD])
    x16 = jnp.pad(x, ((0, npad_extra), (0, 14)))
    batchp = jnp.pad(batch.astype(jnp.int32), (0, npad_extra),
                     constant_values=NUM_GRAPHS).reshape(NPAD, 1)
    ones16 = jnp.ones((B, 16), jnp.float32)
    zeros16 = jnp.zeros((NPAD, 16), jnp.float32)
    zeros32 = jnp.zeros((NPAD, 32), jnp.float32)
    b1r = b1.reshape(1, 64)
    b2r = b2.reshape(1, 128)
    fcbr = fcb.reshape(1, 30)

    dacc = _deg_call(dst2, ones16, zeros16)
    dinv, xs16 = _prep_call(dacc, x16)
    aacc = _agg1_call(src2, dst2, xs16, zeros16)
    hs2 = _hidden_call(aacc, xs16, dinv, W1, b1r)
    hsflat = hs2.reshape(NC * NPAD, 32)
    eacc = _agg2_call(srcoff, dst2, hsflat, zeros32)
    return _head_call(eacc, hs2, dinv, batchp, W2, b2r, fcW, fcbr)


# speedup vs baseline: 26.6508x; 1.0235x over previous
import functools
import jax
import jax.numpy as jnp
from jax import lax
from jax.experimental import pallas as pl
from jax.experimental.pallas import tpu as pltpu, tpu_sc as plsc

N_NODES = 50000
NUM_GRAPHS = 128
NPAD = 50176
DUMMY = NPAD - 1
EPAD = 819200
B = 128
NBT = EPAD // B
NC, NS = 2, 16
ROWS_PER_SUB = NPAD // NS
NB1 = NBT // (NC * NS)
NB2 = NBT // NS

_mesh = lambda: plsc.VectorSubcoreMesh(core_axis_name="c", subcore_axis_name="s")
_SC_PARAMS = pltpu.CompilerParams(use_tc_tiling_on_sc=False)


def _zero_acc(zeros_h, acc, s):
    pltpu.sync_copy(zeros_h.at[pl.ds(s * ROWS_PER_SUB, ROWS_PER_SUB)],
                    acc.at[pl.ds(s * ROWS_PER_SUB, ROWS_PER_SUB)])


def _copy_out(acc, out_h, c, s):
    pltpu.sync_copy(acc.at[pl.ds(s * ROWS_PER_SUB, ROWS_PER_SUB)],
                    out_h.at[c, pl.ds(s * ROWS_PER_SUB, ROWS_PER_SUB)])


K = 8
K2 = 5


def _gs_pipe(tbl_h, acc, sidx, didx, slots, gsem, ssem, nb, k):
    for j in range(k):
        pltpu.async_copy(tbl_h.at[sidx.at[j]], slots.at[j], gsem.at[j])

    def rnd(r, carry):
        b0 = r * k
        for j in range(k):
            pltpu.make_async_copy(tbl_h.at[sidx.at[b0 + j]], slots.at[j],
                                  gsem.at[j]).wait()
            pltpu.async_copy(slots.at[j], acc.at[didx.at[b0 + j]], ssem.at[j],
                             add=True)
        for j in range(k):
            pltpu.make_async_copy(slots.at[j], acc.at[didx.at[b0 + j]],
                                  ssem.at[j]).wait()
            pltpu.async_copy(tbl_h.at[sidx.at[b0 + k + j]], slots.at[j],
                             gsem.at[j])
        return carry

    lax.fori_loop(0, nb // k - 1, rnd, 0)
    b0 = nb - k
    for j in range(k):
        pltpu.make_async_copy(tbl_h.at[sidx.at[b0 + j]], slots.at[j],
                              gsem.at[j]).wait()
        pltpu.async_copy(slots.at[j], acc.at[didx.at[b0 + j]], ssem.at[j],
                         add=True)
    for j in range(k):
        pltpu.make_async_copy(slots.at[j], acc.at[didx.at[b0 + j]],
                              ssem.at[j]).wait()


def _deg_body(dst_h, ones_h, zeros_h, out_h, ones_v, didx_all, acc,
              sem0, sem1):
    c = lax.axis_index("c")
    s = lax.axis_index("s")
    w = c * NS + s
    _zero_acc(zeros_h, acc, s)
    pltpu.sync_copy(ones_h, ones_v)
    pltpu.sync_copy(dst_h.at[pl.ds(w * NB1, NB1)], didx_all)
    plsc.subcore_barrier()

    for j in range(K):
        pltpu.async_copy(ones_v, acc.at[didx_all.at[j]], sem0, add=True)

    def body2(g, carry):
        b0 = 2 * g * K
        for j in range(K):
            pltpu.async_copy(ones_v, acc.at[didx_all.at[b0 + K + j]], sem1,
                             add=True)
        for j in range(K):
            pltpu.make_async_copy(ones_v, acc.at[didx_all.at[b0 + j]],
                                  sem0).wait()
        for j in range(K):
            pltpu.async_copy(ones_v, acc.at[didx_all.at[b0 + 2 * K + j]],
                             sem0, add=True)
        for j in range(K):
            pltpu.make_async_copy(ones_v, acc.at[didx_all.at[b0 + K + j]],
                                  sem1).wait()
        return carry

    ngroups = NB1 // K
    lax.fori_loop(0, (ngroups - 1) // 2, body2, 0)
    b0 = (ngroups - 1) * K
    for j in range(K):
        pltpu.make_async_copy(ones_v, acc.at[didx_all.at[b0 + j]],
                              sem0).wait()
    plsc.subcore_barrier()
    _copy_out(acc, out_h, c, s)


def _deg_call(dst2, ones16, zeros16):
    return pl.kernel(
        _deg_body,
        out_type=jax.ShapeDtypeStruct((NC, NPAD, 16), jnp.float32),
        mesh=_mesh(),
        compiler_params=_SC_PARAMS,
        scratch_types=[
            pltpu.VMEM((B, 16), jnp.float32),
            pltpu.VMEM((NB1, B), jnp.int32),
            pltpu.VMEM_SHARED((NPAD, 16), jnp.float32),
            pltpu.SemaphoreType.DMA,
            pltpu.SemaphoreType.DMA,
        ],
    )(dst2, ones16, zeros16)


CH1 = 40


def _agg1_body(src_h, dst_h, tbl_h, zeros_h, out_h, sidx_ch, didx_ch,
               slots, tbl_s, acc, gsem, ssem):
    c = lax.axis_index("c")
    s = lax.axis_index("s")
    w = c * NS + s
    _zero_acc(zeros_h, acc, s)
    pltpu.sync_copy(tbl_h.at[pl.ds(s * ROWS_PER_SUB, ROWS_PER_SUB)],
                    tbl_s.at[pl.ds(s * ROWS_PER_SUB, ROWS_PER_SUB)])
    plsc.subcore_barrier()

    def chunk(k, carry):
        base = w * NB1 + k * CH1
        pltpu.sync_copy(src_h.at[pl.ds(base, CH1)], sidx_ch)
        pltpu.sync_copy(dst_h.at[pl.ds(base, CH1)], didx_ch)
        _gs_pipe(tbl_s, acc, sidx_ch, didx_ch, slots, gsem, ssem, CH1, K)
        return carry

    lax.fori_loop(0, NB1 // CH1, chunk, 0)
    plsc.subcore_barrier()
    _copy_out(acc, out_h, c, s)


def _agg1_call(src2, dst2, xs16, zeros16):
    return pl.kernel(
        _agg1_body,
        out_type=jax.ShapeDtypeStruct((NC, NPAD, 16), jnp.float32),
        mesh=_mesh(),
        compiler_params=_SC_PARAMS,
        scratch_types=[
            pltpu.VMEM((CH1, B), jnp.int32),
            pltpu.VMEM((CH1, B), jnp.int32),
            pltpu.VMEM((K, B, 16), jnp.float32),
            pltpu.VMEM_SHARED((NPAD, 16), jnp.float32),
            pltpu.VMEM_SHARED((NPAD, 16), jnp.float32),
            pltpu.SemaphoreType.DMA((K,)),
            pltpu.SemaphoreType.DMA((K,)),
        ],
    )(src2, dst2, xs16, zeros16)


CH2 = 20


def _agg2_body(src_h, dst_h, tbl_h, zeros_h, out_h, sidx_ch, didx_ch,
               slots, acc, gsem, ssem):
    c = lax.axis_index("c")
    s = lax.axis_index("s")
    _zero_acc(zeros_h, acc, s)
    plsc.subcore_barrier()

    def chunk(k, carry):
        base = s * NB2 + k * CH2
        pltpu.sync_copy(src_h.at[pl.ds(base, CH2)], sidx_ch)
        pltpu.sync_copy(dst_h.at[pl.ds(base, CH2)], didx_ch)
        _gs_pipe(tbl_h.at[c], acc, sidx_ch, didx_ch, slots, gsem, ssem,
                 CH2, K2)
        return carry

    lax.fori_loop(0, NB2 // CH2, chunk, 0)
    plsc.subcore_barrier()
    _copy_out(acc, out_h, c, s)


def _agg2_call(src2, dst2, hs2, zeros32):
    return pl.kernel(
        _agg2_body,
        out_type=jax.ShapeDtypeStruct((NC, NPAD, 32), jnp.float32),
        mesh=_mesh(),
        compiler_params=_SC_PARAMS,
        scratch_types=[
            pltpu.VMEM((CH2, B), jnp.int32),
            pltpu.VMEM((CH2, B), jnp.int32),
            pltpu.VMEM((K2, B, 32), jnp.float32),
            pltpu.VMEM_SHARED((NPAD, 32), jnp.float32),
            pltpu.SemaphoreType.DMA((K2,)),
            pltpu.SemaphoreType.DMA((K2,)),
        ],
    )(src2, dst2, hs2, zeros32)


RB = 6272
NBLK = NPAD // RB


def _prep_body(dacc, x2, dinv_o, xs16_o):
    deg = dacc[0, :, 0:1] + dacc[1, :, 0:1] + 1.0
    dinv = lax.rsqrt(deg)
    dinv_o[...] = dinv
    xs16_o[...] = jnp.pad(x2[...] * dinv, ((0, 0), (0, 14)))


def _prep_call(dacc, x2):
    return pl.pallas_call(
        _prep_body,
        grid=(NBLK,),
        in_specs=[
            pl.BlockSpec((NC, RB, 16), lambda i: (0, i, 0)),
            pl.BlockSpec((RB, 2), lambda i: (i, 0)),
        ],
        out_specs=[
            pl.BlockSpec((RB, 1), lambda i: (i, 0)),
            pl.BlockSpec((RB, 16), lambda i: (i, 0)),
        ],
        out_shape=[
            jax.ShapeDtypeStruct((NPAD, 1), jnp.float32),
            jax.ShapeDtypeStruct((NPAD, 16), jnp.float32),
        ],
    )(dacc, x2)


def _hidden_body(aacc, xs16, dinv, W1, b1, hs_o):
    s = (aacc[0] + aacc[1] + xs16[...]) * dinv[...]
    h = jnp.maximum(
        jax.lax.dot_general(s[:, 0:2], W1[...], (((1,), (0,)), ((), ())),
                            preferred_element_type=jnp.float32) + b1[...], 0.0)
    hs = h * dinv[...]
    hs_o[0] = hs[:, 0:32]
    hs_o[1] = hs[:, 32:64]


def _hidden_call(aacc, xs16, dinv, W1, b1):
    return pl.pallas_call(
        _hidden_body,
        grid=(NBLK,),
        in_specs=[
            pl.BlockSpec((NC, RB, 16), lambda i: (0, i, 0)),
            pl.BlockSpec((RB, 16), lambda i: (i, 0)),
            pl.BlockSpec((RB, 1), lambda i: (i, 0)),
            pl.BlockSpec((2, 64), lambda i: (0, 0)),
            pl.BlockSpec((1, 64), lambda i: (0, 0)),
        ],
        out_specs=pl.BlockSpec((NC, RB, 32), lambda i: (0, i, 0)),
        out_shape=jax.ShapeDtypeStruct((NC, NPAD, 32), jnp.float32),
    )(aacc, xs16, dinv, W1, b1)


def _head_body(eacc, hs2, dinv, batchp, W2, b2, fcW, fcb, out_o, pooled, cnt):
    i = pl.program_id(0)

    @pl.when(i == 0)
    def _init():
        pooled[...] = jnp.zeros_like(pooled)
        cnt[...] = jnp.zeros_like(cnt)

    a = jnp.concatenate([eacc[0] + hs2[0], eacc[1] + hs2[1]], axis=1)
    a = a * dinv[...]
    oh = (batchp[...] == lax.broadcasted_iota(jnp.int32, (1, NUM_GRAPHS), 1))
    oh = oh.astype(jnp.float32)
    pooled[...] += lax.dot_general(oh, a, (((0,), (0,)), ((), ())),
                                   preferred_element_type=jnp.float32)
    cnt[...] += lax.dot_general(oh, jnp.ones((RB, 1), jnp.float32),
                                (((0,), (0,)), ((), ())),
                                preferred_element_type=jnp.float32)

    @pl.when(i == NBLK - 1)
    def _fin():
        pm = pooled[...] / jnp.maximum(cnt[...], 1.0)
        z = lax.dot_general(pm, W2[...], (((1,), (0,)), ((), ())),
                            preferred_element_type=jnp.float32) + b2[...]
        z = lax.dot_general(z, fcW[...], (((1,), (0,)), ((), ())),
                            preferred_element_type=jnp.float32) + fcb[...]
        m = jnp.max(z, axis=1, keepdims=True)
        out_o[...] = z - m - jnp.log(jnp.sum(jnp.exp(z - m), axis=1,
                                             keepdims=True))


def _head_call(eacc, hs2, dinv, batchp, W2, b2, fcW, fcb):
    return pl.pallas_call(
        _head_body,
        grid=(NBLK,),
        in_specs=[
            pl.BlockSpec((NC, RB, 32), lambda i: (0, i, 0)),
            pl.BlockSpec((NC, RB, 32), lambda i: (0, i, 0)),
            pl.BlockSpec((RB, 1), lambda i: (i, 0)),
            pl.BlockSpec((RB, 1), lambda i: (i, 0)),
            pl.BlockSpec((64, 128), lambda i: (0, 0)),
            pl.BlockSpec((1, 128), lambda i: (0, 0)),
            pl.BlockSpec((128, 30), lambda i: (0, 0)),
            pl.BlockSpec((1, 30), lambda i: (0, 0)),
        ],
        out_specs=pl.BlockSpec((NUM_GRAPHS, 30), lambda i: (0, 0)),
        out_shape=jax.ShapeDtypeStruct((NUM_GRAPHS, 30), jnp.float32),
        scratch_shapes=[
            pltpu.VMEM((NUM_GRAPHS, 64), jnp.float32),
            pltpu.VMEM((NUM_GRAPHS, 1), jnp.float32),
        ],
    )(eacc, hs2, dinv, batchp, W2, b2, fcW, fcb)


@jax.jit
def kernel(x, edge_index, batch, W1, b1, W2, b2, fcW, fcb):
    src = edge_index[0].astype(jnp.int32)
    dst = edge_index[1].astype(jnp.int32)
    npad_extra = NPAD - N_NODES
    epad_extra = EPAD - src.shape[0]
    srcp = jnp.pad(src, (0, epad_extra))
    dstp = jnp.pad(dst, (0, epad_extra), constant_values=DUMMY)
    src2 = srcp.reshape(NBT, B)
    dst2 = dstp.reshape(NBT, B)
    x2 = jnp.pad(x, ((0, npad_extra), (0, 0)))
    batchp = jnp.pad(batch.astype(jnp.int32), (0, npad_extra),
                     constant_values=NUM_GRAPHS).reshape(NPAD, 1)
    ones16 = jnp.ones((B, 16), jnp.float32)
    zeros16 = jnp.zeros((NPAD, 16), jnp.float32)
    zeros32 = jnp.zeros((NPAD, 32), jnp.float32)
    b1r = b1.reshape(1, 64)
    b2r = b2.reshape(1, 128)
    fcbr = fcb.reshape(1, 30)

    dacc = _deg_call(dst2, ones16, zeros16)
    dinv, xs16 = _prep_call(dacc, x2)
    aacc = _agg1_call(src2, dst2, xs16, zeros16)
    hs2 = _hidden_call(aacc, xs16, dinv, W1, b1r)
    eacc = _agg2_call(src2, dst2, hs2, zeros32)
    return _head_call(eacc, hs2, dinv, batchp, W2, b2r, fcW, fcbr)


# speedup vs baseline: 29.6753x; 1.1135x over previous
import functools
import jax
import jax.numpy as jnp
from jax import lax
from jax.experimental import pallas as pl
from jax.experimental.pallas import tpu as pltpu, tpu_sc as plsc

N_NODES = 50000
NUM_GRAPHS = 128
NPAD = 50176
DUMMY = NPAD - 1
EPAD = 819200
B = 128
NBT = EPAD // B
NC, NS = 2, 16
ROWS_PER_SUB = NPAD // NS
NB1 = NBT // (NC * NS)
NB2 = NBT // NS

_mesh = lambda: plsc.VectorSubcoreMesh(core_axis_name="c", subcore_axis_name="s")
_SC_PARAMS = pltpu.CompilerParams(use_tc_tiling_on_sc=False)


def _zero_acc(zeros_h, acc, s):
    pltpu.sync_copy(zeros_h.at[pl.ds(s * ROWS_PER_SUB, ROWS_PER_SUB)],
                    acc.at[pl.ds(s * ROWS_PER_SUB, ROWS_PER_SUB)])


def _copy_out(acc, out_h, c, s):
    pltpu.sync_copy(acc.at[pl.ds(s * ROWS_PER_SUB, ROWS_PER_SUB)],
                    out_h.at[c, pl.ds(s * ROWS_PER_SUB, ROWS_PER_SUB)])


K = 8
K2 = 5


def _gs_pipe(tbl_h, acc, sidx, didx, slots, gsem, ssem, nb, k):
    for j in range(k):
        pltpu.async_copy(tbl_h.at[sidx.at[j]], slots.at[j], gsem.at[j])

    def rnd(r, carry):
        b0 = r * k
        for j in range(k):
            pltpu.make_async_copy(tbl_h.at[sidx.at[b0 + j]], slots.at[j],
                                  gsem.at[j]).wait()
            pltpu.async_copy(slots.at[j], acc.at[didx.at[b0 + j]], ssem.at[j],
                             add=True)
        for j in range(k):
            pltpu.make_async_copy(slots.at[j], acc.at[didx.at[b0 + j]],
                                  ssem.at[j]).wait()
            pltpu.async_copy(tbl_h.at[sidx.at[b0 + k + j]], slots.at[j],
                             gsem.at[j])
        return carry

    lax.fori_loop(0, nb // k - 1, rnd, 0)
    b0 = nb - k
    for j in range(k):
        pltpu.make_async_copy(tbl_h.at[sidx.at[b0 + j]], slots.at[j],
                              gsem.at[j]).wait()
        pltpu.async_copy(slots.at[j], acc.at[didx.at[b0 + j]], ssem.at[j],
                         add=True)
    for j in range(k):
        pltpu.make_async_copy(slots.at[j], acc.at[didx.at[b0 + j]],
                              ssem.at[j]).wait()


def _deg_body(dst_h, ones_h, zeros_h, out_h, ones_v, didx_all, acc,
              sem0, sem1):
    c = lax.axis_index("c")
    s = lax.axis_index("s")
    w = c * NS + s
    _zero_acc(zeros_h, acc, s)
    pltpu.sync_copy(ones_h, ones_v)
    pltpu.sync_copy(dst_h.at[pl.ds(w * NB1, NB1)], didx_all)
    plsc.subcore_barrier()

    for j in range(K):
        pltpu.async_copy(ones_v, acc.at[didx_all.at[j]], sem0, add=True)

    def body2(g, carry):
        b0 = 2 * g * K
        for j in range(K):
            pltpu.async_copy(ones_v, acc.at[didx_all.at[b0 + K + j]], sem1,
                             add=True)
        for j in range(K):
            pltpu.make_async_copy(ones_v, acc.at[didx_all.at[b0 + j]],
                                  sem0).wait()
        for j in range(K):
            pltpu.async_copy(ones_v, acc.at[didx_all.at[b0 + 2 * K + j]],
                             sem0, add=True)
        for j in range(K):
            pltpu.make_async_copy(ones_v, acc.at[didx_all.at[b0 + K + j]],
                                  sem1).wait()
        return carry

    ngroups = NB1 // K
    lax.fori_loop(0, (ngroups - 1) // 2, body2, 0)
    b0 = (ngroups - 1) * K
    for j in range(K):
        pltpu.make_async_copy(ones_v, acc.at[didx_all.at[b0 + j]],
                              sem0).wait()
    plsc.subcore_barrier()
    _copy_out(acc, out_h, c, s)


def _deg_call(dst2, ones16, zeros16):
    return pl.kernel(
        _deg_body,
        out_type=jax.ShapeDtypeStruct((NC, NPAD, 16), jnp.float32),
        mesh=_mesh(),
        compiler_params=_SC_PARAMS,
        scratch_types=[
            pltpu.VMEM((B, 16), jnp.float32),
            pltpu.VMEM((NB1, B), jnp.int32),
            pltpu.VMEM_SHARED((NPAD, 16), jnp.float32),
            pltpu.SemaphoreType.DMA,
            pltpu.SemaphoreType.DMA,
        ],
    )(dst2, ones16, zeros16)


CH1 = 40


def _agg1_body(src_h, dst_h, tbl_h, zeros_h, out_h, sidx_ch, didx_ch,
               slots, tbl_s, acc, gsem, ssem):
    c = lax.axis_index("c")
    s = lax.axis_index("s")
    w = c * NS + s
    _zero_acc(zeros_h, acc, s)
    pltpu.sync_copy(tbl_h.at[pl.ds(s * ROWS_PER_SUB, ROWS_PER_SUB)],
                    tbl_s.at[pl.ds(s * ROWS_PER_SUB, ROWS_PER_SUB)])
    plsc.subcore_barrier()

    def chunk(k, carry):
        base = w * NB1 + k * CH1
        pltpu.sync_copy(src_h.at[pl.ds(base, CH1)], sidx_ch)
        pltpu.sync_copy(dst_h.at[pl.ds(base, CH1)], didx_ch)
        _gs_pipe(tbl_s, acc, sidx_ch, didx_ch, slots, gsem, ssem, CH1, K)
        return carry

    lax.fori_loop(0, NB1 // CH1, chunk, 0)
    plsc.subcore_barrier()
    _copy_out(acc, out_h, c, s)


def _agg1_call(src2, dst2, xs16, zeros16):
    return pl.kernel(
        _agg1_body,
        out_type=jax.ShapeDtypeStruct((NC, NPAD, 16), jnp.float32),
        mesh=_mesh(),
        compiler_params=_SC_PARAMS,
        scratch_types=[
            pltpu.VMEM((CH1, B), jnp.int32),
            pltpu.VMEM((CH1, B), jnp.int32),
            pltpu.VMEM((K, B, 16), jnp.float32),
            pltpu.VMEM_SHARED((NPAD, 16), jnp.float32),
            pltpu.VMEM_SHARED((NPAD, 16), jnp.float32),
            pltpu.SemaphoreType.DMA((K,)),
            pltpu.SemaphoreType.DMA((K,)),
        ],
    )(src2, dst2, xs16, zeros16)


CH2 = 40


def _agg2_body(src_h, dst_h, tbl_h, zeros_h, out_h, sidx_ch, didx_ch,
               slots, tbl_s, acc, gsem, ssem):
    c = lax.axis_index("c")
    s = lax.axis_index("s")
    for g in range(2):
        grp = c * 2 + g
        _zero_acc(zeros_h, acc, s)
        pltpu.sync_copy(tbl_h.at[grp, pl.ds(s * ROWS_PER_SUB, ROWS_PER_SUB)],
                        tbl_s.at[pl.ds(s * ROWS_PER_SUB, ROWS_PER_SUB)])
        plsc.subcore_barrier()

        def chunk(k, carry):
            base = s * NB2 + k * CH2
            pltpu.sync_copy(src_h.at[pl.ds(base, CH2)], sidx_ch)
            pltpu.sync_copy(dst_h.at[pl.ds(base, CH2)], didx_ch)
            _gs_pipe(tbl_s, acc, sidx_ch, didx_ch, slots, gsem, ssem, CH2, K)
            return carry

        lax.fori_loop(0, NB2 // CH2, chunk, 0)
        plsc.subcore_barrier()
        pltpu.sync_copy(acc.at[pl.ds(s * ROWS_PER_SUB, ROWS_PER_SUB)],
                        out_h.at[grp, pl.ds(s * ROWS_PER_SUB, ROWS_PER_SUB)])
        plsc.subcore_barrier()


def _agg2_call(src2, dst2, hs4, zeros16):
    return pl.kernel(
        _agg2_body,
        out_type=jax.ShapeDtypeStruct((4, NPAD, 16), jnp.float32),
        mesh=_mesh(),
        compiler_params=_SC_PARAMS,
        scratch_types=[
            pltpu.VMEM((CH2, B), jnp.int32),
            pltpu.VMEM((CH2, B), jnp.int32),
            pltpu.VMEM((K, B, 16), jnp.float32),
            pltpu.VMEM_SHARED((NPAD, 16), jnp.float32),
            pltpu.VMEM_SHARED((NPAD, 16), jnp.float32),
            pltpu.SemaphoreType.DMA((K,)),
            pltpu.SemaphoreType.DMA((K,)),
        ],
    )(src2, dst2, hs4, zeros16)


RB = 6272
NBLK = NPAD // RB


def _prep_body(dacc, x2, dinv_o, xs16_o):
    deg = dacc[0, :, 0:1] + dacc[1, :, 0:1] + 1.0
    dinv = lax.rsqrt(deg)
    dinv_o[...] = dinv
    xs16_o[...] = jnp.pad(x2[...] * dinv, ((0, 0), (0, 14)))


def _prep_call(dacc, x2):
    return pl.pallas_call(
        _prep_body,
        grid=(NBLK,),
        in_specs=[
            pl.BlockSpec((NC, RB, 16), lambda i: (0, i, 0)),
            pl.BlockSpec((RB, 2), lambda i: (i, 0)),
        ],
        out_specs=[
            pl.BlockSpec((RB, 1), lambda i: (i, 0)),
            pl.BlockSpec((RB, 16), lambda i: (i, 0)),
        ],
        out_shape=[
            jax.ShapeDtypeStruct((NPAD, 1), jnp.float32),
            jax.ShapeDtypeStruct((NPAD, 16), jnp.float32),
        ],
    )(dacc, x2)


def _hidden_body(aacc, xs16, dinv, W1, b1, hs_o):
    s = (aacc[0] + aacc[1] + xs16[...]) * dinv[...]
    h = jnp.maximum(
        jax.lax.dot_general(s[:, 0:2], W1[...], (((1,), (0,)), ((), ())),
                            preferred_element_type=jnp.float32) + b1[...], 0.0)
    hs = h * dinv[...]
    hs_o[0] = hs[:, 0:16]
    hs_o[1] = hs[:, 16:32]
    hs_o[2] = hs[:, 32:48]
    hs_o[3] = hs[:, 48:64]


def _hidden_call(aacc, xs16, dinv, W1, b1):
    return pl.pallas_call(
        _hidden_body,
        grid=(NBLK,),
        in_specs=[
            pl.BlockSpec((NC, RB, 16), lambda i: (0, i, 0)),
            pl.BlockSpec((RB, 16), lambda i: (i, 0)),
            pl.BlockSpec((RB, 1), lambda i: (i, 0)),
            pl.BlockSpec((2, 64), lambda i: (0, 0)),
            pl.BlockSpec((1, 64), lambda i: (0, 0)),
        ],
        out_specs=pl.BlockSpec((4, RB, 16), lambda i: (0, i, 0)),
        out_shape=jax.ShapeDtypeStruct((4, NPAD, 16), jnp.float32),
    )(aacc, xs16, dinv, W1, b1)


RBH = 1568
NBH = NPAD // RBH


def _head_body(eacc, hs2, dinv, batchp, W2, b2, fcW, fcb, out_o, pooled, cnt):
    i = pl.program_id(0)

    @pl.when(i == 0)
    def _init():
        pooled[...] = jnp.zeros_like(pooled)
        cnt[...] = jnp.zeros_like(cnt)

    a = jnp.concatenate([eacc[0] + hs2[0], eacc[1] + hs2[1],
                         eacc[2] + hs2[2], eacc[3] + hs2[3]], axis=1)
    a = a * dinv[...]
    oh = (batchp[...] == lax.broadcasted_iota(jnp.int32, (1, NUM_GRAPHS), 1))
    oh = oh.astype(jnp.float32)
    pooled[...] += lax.dot_general(oh, a, (((0,), (0,)), ((), ())),
                                   preferred_element_type=jnp.float32)
    cnt[...] += lax.dot_general(oh, jnp.ones((RBH, 1), jnp.float32),
                                (((0,), (0,)), ((), ())),
                                preferred_element_type=jnp.float32)

    @pl.when(i == NBH - 1)
    def _fin():
        pm = pooled[...] / jnp.maximum(cnt[...], 1.0)
        z = lax.dot_general(pm, W2[...], (((1,), (0,)), ((), ())),
                            preferred_element_type=jnp.float32) + b2[...]
        z = lax.dot_general(z, fcW[...], (((1,), (0,)), ((), ())),
                            preferred_element_type=jnp.float32) + fcb[...]
        m = jnp.max(z, axis=1, keepdims=True)
        out_o[...] = z - m - jnp.log(jnp.sum(jnp.exp(z - m), axis=1,
                                             keepdims=True))


def _head_call(eacc, hs2, dinv, batchp, W2, b2, fcW, fcb):
    return pl.pallas_call(
        _head_body,
        grid=(NBH,),
        in_specs=[
            pl.BlockSpec((4, RBH, 16), lambda i: (0, i, 0)),
            pl.BlockSpec((4, RBH, 16), lambda i: (0, i, 0)),
            pl.BlockSpec((RBH, 1), lambda i: (i, 0)),
            pl.BlockSpec((RBH, 1), lambda i: (i, 0)),
            pl.BlockSpec((64, 128), lambda i: (0, 0)),
            pl.BlockSpec((1, 128), lambda i: (0, 0)),
            pl.BlockSpec((128, 30), lambda i: (0, 0)),
            pl.BlockSpec((1, 30), lambda i: (0, 0)),
        ],
        out_specs=pl.BlockSpec((NUM_GRAPHS, 30), lambda i: (0, 0)),
        out_shape=jax.ShapeDtypeStruct((NUM_GRAPHS, 30), jnp.float32),
        scratch_shapes=[
            pltpu.VMEM((NUM_GRAPHS, 64), jnp.float32),
            pltpu.VMEM((NUM_GRAPHS, 1), jnp.float32),
        ],
    )(eacc, hs2, dinv, batchp, W2, b2, fcW, fcb)


@jax.jit
def kernel(x, edge_index, batch, W1, b1, W2, b2, fcW, fcb):
    src = edge_index[0].astype(jnp.int32)
    dst = edge_index[1].astype(jnp.int32)
    npad_extra = NPAD - N_NODES
    epad_extra = EPAD - src.shape[0]
    srcp = jnp.pad(src, (0, epad_extra))
    dstp = jnp.pad(dst, (0, epad_extra), constant_values=DUMMY)
    src2 = srcp.reshape(NBT, B)
    dst2 = dstp.reshape(NBT, B)
    x2 = jnp.pad(x, ((0, npad_extra), (0, 0)))
    batchp = jnp.pad(batch.astype(jnp.int32), (0, npad_extra),
                     constant_values=NUM_GRAPHS).reshape(NPAD, 1)
    ones16 = jnp.ones((B, 16), jnp.float32)
    zeros16 = jnp.zeros((NPAD, 16), jnp.float32)
    b1r = b1.reshape(1, 64)
    b2r = b2.reshape(1, 128)
    fcbr = fcb.reshape(1, 30)

    dacc = _deg_call(dst2, ones16, zeros16)
    dinv, xs16 = _prep_call(dacc, x2)
    aacc = _agg1_call(src2, dst2, xs16, zeros16)
    hs4 = _hidden_call(aacc, xs16, dinv, W1, b1r)
    eacc = _agg2_call(src2, dst2, hs4, zeros16)
    return _head_call(eacc, hs4, dinv, batchp, W2, b2r, fcW, fcbr)
